# Initial kernel scaffold; baseline (speedup 1.0000x reference)
import jax
import jax.numpy as jnp
from jax.experimental import pallas as pl


def kernel(x, emb):
    raise NotImplementedError("write your pallas kernel here")


# speedup vs baseline: 1.6418x; 1.6418x over previous
import functools

import jax
import jax.numpy as jnp
from jax import lax
from jax.experimental import pallas as pl
from jax.experimental.pallas import tpu as pltpu
from jax.experimental.pallas import tpu_sc as plsc


def _broadcast_emb(B, S, D, dtype):
    info = plsc.get_sparse_core_info()
    nw = info.num_cores * info.num_subcores
    rows_per_w = S // nw
    chunk = 64
    n_chunks = rows_per_w // chunk
    mesh = plsc.VectorSubcoreMesh(core_axis_name="c", subcore_axis_name="s")

    @functools.partial(
        pl.kernel,
        mesh=mesh,
        out_type=jax.ShapeDtypeStruct((B, S, D), dtype),
        scratch_types=[
            pltpu.VMEM((chunk, D), dtype),
            pltpu.SemaphoreType.DMA,
        ],
    )
    def k(emb_hbm, out_hbm, buf, sem):
        wid = lax.axis_index("s") * info.num_cores + lax.axis_index("c")
        base = wid * rows_per_w
        for i in range(n_chunks):
            r0 = base + i * chunk
            pltpu.sync_copy(emb_hbm.at[pl.ds(r0, chunk), :], buf)
            copies = [
                pltpu.async_copy(buf, out_hbm.at[b, pl.ds(r0, chunk), :], sem)
                for b in range(B)
            ]
            for c in copies:
                c.wait()

    return k


def kernel(x, emb):
    B, S, D = x.shape
    return _broadcast_emb(B, S, D, emb.dtype)(emb)
